# baseline (speedup 1.0000x reference)
import functools

import jax
import jax.numpy as jnp
from jax import lax
from jax.experimental import pallas as pl
from jax.experimental.pallas import tpu as pltpu
from jax.experimental.pallas import tpu_sc as plsc

L = 16
NW = 32
CH = 128


def _gate_matvec(h, w2, b2):
    def body(h_ref, w_ref, b_ref, o_ref):
        o_ref[...] = lax.dot_general(
            w_ref[...], h_ref[...],
            dimension_numbers=(((1,), (1,)), ((), ())),
            preferred_element_type=jnp.float32) + b_ref[...]
    return pl.pallas_call(
        body,
        out_shape=jax.ShapeDtypeStruct((2, h.shape[0]), jnp.float32),
    )(h, w2, b2)


def _combine(part, n):
    _, _, d = part.shape
    br = 2000
    def body(p_ref, o_ref):
        o_ref[...] = p_ref[0] + p_ref[1]
    return pl.pallas_call(
        body,
        grid=(n // br,),
        in_specs=[pl.BlockSpec((2, br, d), lambda i: (0, i, 0))],
        out_specs=pl.BlockSpec((br, d), lambda i: (i, 0)),
        out_shape=jax.ShapeDtypeStruct((n, d), jnp.float32),
    )(part)


@functools.cache
def _make_sc(N, D, NCH, NPAD):
    nj = D // L
    z_rows_pt = NPAD // L
    mesh = plsc.VectorSubcoreMesh(core_axis_name="c", subcore_axis_name="s")

    @functools.partial(
        pl.kernel,
        out_type=jax.ShapeDtypeStruct((2, NPAD, D), jnp.float32),
        mesh=mesh,
        compiler_params=pltpu.CompilerParams(needs_layout_passes=False),
        scratch_types=[
            pltpu.VMEM((3, CH), jnp.int32),
            pltpu.VMEM((3, CH), jnp.int32),
            pltpu.VMEM((CH, D), jnp.float32),
            pltpu.VMEM((CH, D), jnp.float32),
            pltpu.VMEM((CH,), jnp.int32),
            pltpu.VMEM((CH,), jnp.int32),
            pltpu.VMEM((CH,), jnp.float32),
            pltpu.VMEM((CH,), jnp.float32),
            pltpu.VMEM((CH,), jnp.float32),
            pltpu.VMEM((CH,), jnp.float32),
            pltpu.VMEM((CH,), jnp.float32),
            pltpu.VMEM((CH,), jnp.float32),
            pltpu.VMEM((CH,), jnp.float32),
            pltpu.VMEM((CH,), jnp.float32),
            pltpu.VMEM_SHARED((NPAD, D), jnp.float32),
            pltpu.VMEM_SHARED((N,), jnp.float32),
            pltpu.VMEM_SHARED((N,), jnp.float32),
            pltpu.VMEM_SHARED((N,), jnp.float32),
            pltpu.SemaphoreType.DMA,
            pltpu.SemaphoreType.DMA,
            pltpu.SemaphoreType.DMA,
            pltpu.SemaphoreType.DMA,
            pltpu.SemaphoreType.DMA,
            pltpu.SemaphoreType.DMA,
            pltpu.SemaphoreType.DMA,
            pltpu.SemaphoreType.DMA,
        ],
    )
    def sc_fn(h_hbm, a_hbm, b_hbm, d_hbm, ed_hbm, z0_hbm, out_hbm,
              eb0, eb1, rw0, rw1, tb0, tb1,
              ga0, ga1, gb0, gb1, gc0, gc1, gd0, gd1,
              z_sh, a_sh, b_sh, d_sh,
              se0, se1, sg0, sg1, ss0, ss1, sx0, sx1):
        cid = lax.axis_index("c")
        sid = lax.axis_index("s")
        wid = sid * 2 + cid

        ebufs, rowss, tbufs = (eb0, eb1), (rw0, rw1), (tb0, tb1)
        gas, gbs, gcs, gds = (ga0, ga1), (gb0, gb1), (gc0, gc1), (gd0, gd1)
        ses, sgs, sss = (se0, se1), (sg0, sg1), (ss0, ss1)
        sxs = (sx0, sx1)

        @pl.when(sid == 0)
        def _():
            pltpu.sync_copy(a_hbm, a_sh)
            pltpu.sync_copy(b_hbm, b_sh)
            pltpu.sync_copy(d_hbm, d_sh)
        pltpu.sync_copy(z0_hbm.at[pl.ds(sid * z_rows_pt, z_rows_pt)],
                        z_sh.at[pl.ds(sid * z_rows_pt, z_rows_pt)])
        plsc.subcore_barrier()

        iota = lax.iota(jnp.int32, L)
        base_q = wid * NCH

        def start_e(q, slot):
            pltpu.async_copy(ed_hbm.at[q], ebufs[slot], ses[slot])

        def wait_e(slot):
            pltpu.make_async_copy(ed_hbm.at[0], ebufs[slot], ses[slot]).wait()

        def start_g(slot):
            pltpu.async_copy(h_hbm.at[ebufs[slot].at[0]], rowss[slot],
                             sgs[slot])

        def wait_g(slot):
            pltpu.make_async_copy(h_hbm.at[ebufs[slot].at[0]], rowss[slot],
                                  sgs[slot]).wait()

        def start_s(slot):
            pltpu.async_copy(rowss[slot], z_sh.at[tbufs[slot]], sss[slot],
                             add=True)

        def wait_s(slot):
            pltpu.make_async_copy(rowss[slot], z_sh.at[tbufs[slot]],
                                  sss[slot]).wait()

        def start_x(slot):
            eb = ebufs[slot]
            pltpu.async_copy(a_sh.at[eb.at[1]], gas[slot], sxs[slot])
            pltpu.async_copy(b_sh.at[eb.at[0]], gbs[slot], sxs[slot])
            pltpu.async_copy(d_sh.at[eb.at[1]], gcs[slot], sxs[slot])
            pltpu.async_copy(d_sh.at[eb.at[0]], gds[slot], sxs[slot])

        def wait_x(slot):
            eb = ebufs[slot]
            pltpu.make_async_copy(a_sh.at[eb.at[1]], gas[slot],
                                  sxs[slot]).wait()
            pltpu.make_async_copy(b_sh.at[eb.at[0]], gbs[slot],
                                  sxs[slot]).wait()
            pltpu.make_async_copy(d_sh.at[eb.at[1]], gcs[slot],
                                  sxs[slot]).wait()
            pltpu.make_async_copy(d_sh.at[eb.at[0]], gds[slot],
                                  sxs[slot]).wait()

        def compute(slot):
            eb, rv, tb = ebufs[slot], rowss[slot], tbufs[slot]
            wait_x(slot)

            def group(g, carry2):
                sl = pl.ds(g * L, L)
                t16 = eb[1, sl]
                w16 = plsc.bitcast(eb[2, sl], jnp.float32)
                tb[sl] = t16
                a_t = gas[slot][sl]
                b_s = gbs[slot][sl]
                d_t = gcs[slot][sl]
                d_s = gds[slot][sl]
                ex = jnp.exp((a_t + b_s) * 2.0)
                coef = (1.0 - 2.0 / (ex + 1.0)) * (d_t * d_s * w16)
                rbase = g * L + iota

                def jloop(j, carry3):
                    cbase = j * L
                    for k in range(L):
                        ccol = cbase + ((iota + k) & (L - 1))
                        v = plsc.load_gather(rv, [rbase, ccol])
                        plsc.store_scatter(rv, [rbase, ccol], v * coef)
                    return carry3

                lax.fori_loop(0, nj, jloop, 0)
                return carry2

            lax.fori_loop(0, CH // L, group, 0)

        start_e(base_q, 0)
        start_e(base_q + 1, 1)
        wait_e(0)
        start_g(0)
        start_x(0)

        def pair(i, carry):
            for slot in (0, 1):
                c = 2 * i + slot
                wait_g(slot)
                compute(slot)
                start_e(c + 2 + base_q, slot)
                if slot == 0:
                    @pl.when(i >= 1)
                    def _():
                        wait_s(1)
                else:
                    wait_s(0)
                wait_e(1 - slot)
                start_g(1 - slot)
                start_x(1 - slot)
                start_s(slot)
            return carry

        lax.fori_loop(0, NCH // 2, pair, 0)
        wait_g(0)
        wait_x(0)
        wait_e(1)
        wait_s(1)

        plsc.subcore_barrier()
        pltpu.sync_copy(z_sh.at[pl.ds(sid * z_rows_pt, z_rows_pt)],
                        out_hbm.at[cid, pl.ds(sid * z_rows_pt, z_rows_pt)])

    return sc_fn


def kernel(h, edge_index, d, w, gate_w, gate_b):
    N, D = h.shape
    E = edge_index.shape[1]
    src = edge_index[0].astype(jnp.int32)
    dst = edge_index[1].astype(jnp.int32)

    w2 = jnp.concatenate([gate_w[:, :D], gate_w[:, D:]], axis=0)
    b2 = jnp.stack([gate_b, jnp.zeros_like(gate_b)], axis=0)
    ab = _gate_matvec(h, w2, b2)

    nch = 2 * (-(-E // (NW * CH * 2)))
    e_pad = NW * CH * nch
    npad = -(-(N + 1) // 128) * 128
    pad = e_pad - E
    src_p = jnp.concatenate([src, jnp.zeros((pad,), jnp.int32)])
    dst_p = jnp.concatenate([dst, jnp.full((pad,), N, jnp.int32)])
    w_p = jnp.concatenate([w, jnp.zeros((pad,), jnp.float32)])
    w_bits = lax.bitcast_convert_type(w_p, jnp.int32)
    edata = jnp.stack([src_p.reshape(-1, CH), dst_p.reshape(-1, CH),
                       w_bits.reshape(-1, CH)], axis=1)
    edata = jnp.concatenate(
        [edata, jnp.zeros((2, 3, CH), jnp.int32)], axis=0)
    z0 = jnp.zeros((npad, D), jnp.float32)

    part = _make_sc(N, D, nch, npad)(h, ab[0], ab[1], d, edata, z0)
    return _combine(part, N)


# speedup vs baseline: 1.0732x; 1.0732x over previous
import functools

import jax
import jax.numpy as jnp
from jax import lax
from jax.experimental import pallas as pl
from jax.experimental.pallas import tpu as pltpu
from jax.experimental.pallas import tpu_sc as plsc

L = 16
NW = 32
CH = 128
SCAN = 1024
PCAP = SCAN + 2 * CH + L


def _gate_matvec(h, w2, b2):
    def body(h_ref, w_ref, b_ref, o_ref):
        o_ref[...] = lax.dot_general(
            w_ref[...], h_ref[...],
            dimension_numbers=(((1,), (1,)), ((), ())),
            preferred_element_type=jnp.float32) + b_ref[...]
    return pl.pallas_call(
        body,
        out_shape=jax.ShapeDtypeStruct((2, h.shape[0]), jnp.float32),
    )(h, w2, b2)


def _combine(part, n):
    _, _, d = part.shape
    br = 2000
    def body(p_ref, o_ref):
        o_ref[...] = p_ref[0] + p_ref[1]
    return pl.pallas_call(
        body,
        grid=(n // br,),
        in_specs=[pl.BlockSpec((2, br, d), lambda i: (0, i, 0))],
        out_specs=pl.BlockSpec((br, d), lambda i: (i, 0)),
        out_shape=jax.ShapeDtypeStruct((n, d), jnp.float32),
    )(part)


@functools.cache
def _make_sc(N, D, NQSC, NPAD):
    nj = D // L
    ZR = NPAD // L
    mesh = plsc.VectorSubcoreMesh(core_axis_name="c", subcore_axis_name="s")

    @functools.partial(
        pl.kernel,
        out_type=jax.ShapeDtypeStruct((2, NPAD, D), jnp.float32),
        mesh=mesh,
        compiler_params=pltpu.CompilerParams(needs_layout_passes=False),
        scratch_types=[
            pltpu.VMEM((3, SCAN), jnp.int32),
            pltpu.VMEM((3, SCAN), jnp.int32),
            pltpu.VMEM((PCAP,), jnp.int32),
            pltpu.VMEM((PCAP,), jnp.int32),
            pltpu.VMEM((PCAP,), jnp.int32),
            pltpu.VMEM((CH,), jnp.int32),
            pltpu.VMEM((CH,), jnp.int32),
            pltpu.VMEM((CH,), jnp.int32),
            pltpu.VMEM((CH,), jnp.int32),
            pltpu.VMEM((CH,), jnp.int32),
            pltpu.VMEM((CH,), jnp.int32),
            pltpu.VMEM((CH, D), jnp.float32),
            pltpu.VMEM((CH, D), jnp.float32),
            pltpu.VMEM((CH,), jnp.float32),
            pltpu.VMEM((CH,), jnp.float32),
            pltpu.VMEM((CH,), jnp.float32),
            pltpu.VMEM((CH,), jnp.float32),
            pltpu.VMEM((CH,), jnp.float32),
            pltpu.VMEM((CH,), jnp.float32),
            pltpu.VMEM((CH,), jnp.float32),
            pltpu.VMEM((CH,), jnp.float32),
            pltpu.VMEM((ZR, D), jnp.float32),
            pltpu.VMEM_SHARED((N,), jnp.float32),
            pltpu.VMEM_SHARED((N,), jnp.float32),
            pltpu.VMEM_SHARED((N,), jnp.float32),
            pltpu.SemaphoreType.DMA,
            pltpu.SemaphoreType.DMA,
            pltpu.SemaphoreType.DMA,
            pltpu.SemaphoreType.DMA,
            pltpu.SemaphoreType.DMA,
            pltpu.SemaphoreType.DMA,
        ],
    )
    def sc_fn(h_hbm, a_hbm, b_hbm, d_hbm, ed_hbm, out_hbm,
              eb0, eb1, pend_s, pend_d, pend_w,
              bs0, bs1, bd0, bd1, bw0, bw1, rw0, rw1,
              ga0, ga1, gb0, gb1, gc0, gc1, gd0, gd1,
              zl, a_sh, b_sh, d_sh, se0, se1, sb0, sb1, sx0, sx1):
        cid = lax.axis_index("c")
        sid = lax.axis_index("s")

        ebufs, ses, sbs = (eb0, eb1), (se0, se1), (sb0, sb1)
        sxs = (sx0, sx1)
        bss, bds, bws = (bs0, bs1), (bd0, bd1), (bw0, bw1)
        rowss = (rw0, rw1)
        gas, gbs, gcs, gds = (ga0, ga1), (gb0, gb1), (gc0, gc1), (gd0, gd1)

        lo = sid * ZR
        iota = lax.iota(jnp.int32, L)
        zvec = jnp.zeros((L,), jnp.float32)

        @pl.when(sid == 0)
        def _():
            pltpu.sync_copy(a_hbm, a_sh)
            pltpu.sync_copy(b_hbm, b_sh)
            pltpu.sync_copy(d_hbm, d_sh)

        def zrow(r, carry):
            for j in range(nj):
                zl[r, pl.ds(j * L, L)] = zvec
            return carry
        lax.fori_loop(0, ZR, zrow, 0)
        plsc.subcore_barrier()

        base_q = cid * NQSC

        def start_e(q, slot):
            pltpu.async_copy(ed_hbm.at[q], ebufs[slot], ses[slot])

        def wait_e(slot):
            pltpu.make_async_copy(ed_hbm.at[0], ebufs[slot],
                                  ses[slot]).wait()

        def issue(slot, base):
            bs, bd, bw = bss[slot], bds[slot], bws[slot]
            for g in range(CH // L):
                sp = pl.ds(base + g * L, L)
                sb = pl.ds(g * L, L)
                bs[sb] = pend_s[sp]
                bd[sb] = pend_d[sp]
                bw[sb] = pend_w[sp]
            sem = sbs[slot]
            pltpu.async_copy(h_hbm.at[bs], rowss[slot], sem)
            sx = sxs[slot]
            pltpu.async_copy(a_sh.at[bd], gas[slot], sx)
            pltpu.async_copy(b_sh.at[bs], gbs[slot], sx)
            pltpu.async_copy(d_sh.at[bd], gcs[slot], sx)
            pltpu.async_copy(d_sh.at[bs], gds[slot], sx)

        def compute(slot):
            bs, bd, bw = bss[slot], bds[slot], bws[slot]
            sem = sbs[slot]
            pltpu.make_async_copy(h_hbm.at[bs], rowss[slot], sem).wait()
            sx = sxs[slot]
            pltpu.make_async_copy(a_sh.at[bd], gas[slot], sx).wait()
            pltpu.make_async_copy(b_sh.at[bs], gbs[slot], sx).wait()
            pltpu.make_async_copy(d_sh.at[bd], gcs[slot], sx).wait()
            pltpu.make_async_copy(d_sh.at[bs], gds[slot], sx).wait()
            rv = rowss[slot]

            def group(g, carry):
                sl = pl.ds(g * L, L)
                a_t = gas[slot][sl]
                b_s = gbs[slot][sl]
                d_t = gcs[slot][sl]
                d_s = gds[slot][sl]
                w16 = plsc.bitcast(bw[sl], jnp.float32)
                dloc = bd[sl] - lo
                ex = jnp.exp((a_t + b_s) * 2.0)
                coef = (1.0 - 2.0 / (ex + 1.0)) * (d_t * d_s * w16)
                rbase = g * L + iota

                def jloop(j, c2):
                    cbase = j * L
                    for k in range(L):
                        ccol = cbase + ((iota + k) & (L - 1))
                        v = plsc.load_gather(rv, [rbase, ccol])
                        plsc.addupdate_scatter(zl, [dloc, ccol], v * coef)
                    return c2

                lax.fori_loop(0, nj, jloop, 0)
                return carry

            lax.fori_loop(0, CH // L, group, 0)

        def sync_one(slot):
            def body(cc):
                issue(slot, cc - CH)
                compute(slot)
                return cc - CH
            return body

        def scan_group(eb):
            def group(g, cnt):
                sl = pl.ds(g * L, L)
                s16 = eb[0, sl]
                t16 = eb[1, sl]
                wv = eb[2, sl]
                m = (t16 >= lo) & (t16 < lo + ZR)
                plsc.store_compressed(pend_s.at[pl.ds(cnt, L)], s16, mask=m)
                plsc.store_compressed(pend_d.at[pl.ds(cnt, L)], t16, mask=m)
                plsc.store_compressed(pend_w.at[pl.ds(cnt, L)], wv, mask=m)
                pc = plsc.all_reduce_population_count(m)
                return cnt + pc[0]
            return group

        def subchunk(c, slot, cnt, p):
            wait_e(slot)
            cnt = lax.fori_loop(0, SCAN // L, scan_group(ebufs[slot]), cnt)
            start_e(base_q + c + 2, slot)

            def comp_fn():
                compute(slot)
                return jnp.int32(0)
            lax.cond(p == 1, comp_fn, lambda: jnp.int32(0))

            def issue_fn(cc):
                issue(slot, cc - CH)
                return cc - CH, jnp.int32(1)
            cnt, p = lax.cond(cnt >= CH, issue_fn,
                              lambda cc: (cc, jnp.int32(0)), cnt)

            def heavy(args):
                cc, _ = args
                compute(slot)
                cc = lax.while_loop(lambda x: x >= CH, sync_one(slot), cc)
                return cc, jnp.int32(0)
            cnt, p = lax.cond(cnt >= CH, heavy, lambda a: a, (cnt, p))
            return cnt, p

        start_e(base_q, 0)
        start_e(base_q + 1, 1)

        def pairs(i, carry):
            cnt, p0, p1 = carry
            cnt, p0 = subchunk(2 * i, 0, cnt, p0)
            cnt, p1 = subchunk(2 * i + 1, 1, cnt, p1)
            return cnt, p0, p1

        cnt, p0, p1 = lax.fori_loop(0, NQSC // 2, pairs, (
            jnp.int32(0), jnp.int32(0), jnp.int32(0)))

        wait_e(0)
        wait_e(1)

        @pl.when(p0 == 1)
        def _():
            compute(0)

        @pl.when(p1 == 1)
        def _():
            compute(1)

        cnt = lax.while_loop(lambda x: x >= CH, sync_one(0), cnt)

        @pl.when(cnt > 0)
        def _():
            for g in range(CH // L):
                sp = pl.ds(cnt + g * L, L)
                pend_s[sp] = iota * 0
                pend_d[sp] = iota * 0 + lo
                pend_w[sp] = iota * 0
            issue(0, jnp.int32(0))
            compute(0)

        pltpu.sync_copy(zl, out_hbm.at[cid, pl.ds(sid * ZR, ZR)])

    return sc_fn


def kernel(h, edge_index, d, w, gate_w, gate_b):
    N, D = h.shape
    E = edge_index.shape[1]
    src = edge_index[0].astype(jnp.int32)
    dst = edge_index[1].astype(jnp.int32)

    w2 = jnp.concatenate([gate_w[:, :D], gate_w[:, D:]], axis=0)
    b2 = jnp.stack([gate_b, jnp.zeros_like(gate_b)], axis=0)
    ab = _gate_matvec(h, w2, b2)

    e_pad = -(-E // (4 * SCAN)) * (4 * SCAN)
    nqsc = e_pad // SCAN // 2
    npad = -(-(N + 1) // 128) * 128
    pad = e_pad - E
    src_p = jnp.concatenate([src, jnp.zeros((pad,), jnp.int32)])
    dst_p = jnp.concatenate(
        [dst, (jnp.arange(pad, dtype=jnp.int32) * 61) % N])
    w_p = jnp.concatenate([w, jnp.zeros((pad,), jnp.float32)])
    w_bits = lax.bitcast_convert_type(w_p, jnp.int32)
    edata = jnp.stack([src_p.reshape(-1, SCAN), dst_p.reshape(-1, SCAN),
                       w_bits.reshape(-1, SCAN)], axis=1)
    edata = jnp.concatenate(
        [edata, jnp.zeros((2, 3, SCAN), jnp.int32)], axis=0)

    part = _make_sc(N, D, nqsc, npad)(h, ab[0], ab[1], d, edata)
    return _combine(part, N)
